# Initial kernel scaffold; baseline (speedup 1.0000x reference)
import functools
import jax
import jax.numpy as jnp
from jax import lax
from jax.experimental import pallas as pl
from jax.experimental.pallas import tpu as pltpu

NUM_VERTS = 2562
SPS = 500
SPS_PAD = 512
B, S, P = 4, 8, 8
N = 4096
M_PAD = S * SPS_PAD
K = 3
BN = 256
NB = N // BN
PAD_COORD = 3.0e4
BIGF = 3.0e38


def _prep_kernel(a0, a1, a2, t0, t1, t2, w, scl, offs24, sph24,
                 opx, opy, opz, opn, orep):
    a, b, c = a0[...], a1[...], a2[...]
    ca, sa = jnp.cos(a), jnp.sin(a)
    cb, sb = jnp.cos(b), jnp.sin(b)
    cc, sc_ = jnp.cos(c), jnp.sin(c)
    R00 = cb * cc
    R01 = -cb * sc_
    R02 = sb
    R10 = sa * sb * cc + ca * sc_
    R11 = -sa * sb * sc_ + ca * cc
    R12 = -sa * cb
    R20 = -ca * sb * cc + sa * sc_
    R21 = ca * sb * sc_ + sa * cc
    R22 = ca * cb
    wv = w[...]
    ws = wv * scl[...]
    cx = jnp.sum(wv * t0[...], axis=1, keepdims=True)
    cy = jnp.sum(wv * t1[...], axis=1, keepdims=True)
    cz = jnp.sum(wv * t2[...], axis=1, keepdims=True)
    A24x = jnp.concatenate([ws * R00, ws * R01, ws * R02], axis=1)
    A24y = jnp.concatenate([ws * R10, ws * R11, ws * R12], axis=1)
    A24z = jnp.concatenate([ws * R20, ws * R21, ws * R22], axis=1)
    d24 = offs24[...] + sph24[...]
    dbar = jnp.mean(d24, axis=1, keepdims=True)
    d24p = d24[:, :SPS_PAD]
    px = jnp.dot(A24x, d24p, preferred_element_type=jnp.float32) + cx
    py = jnp.dot(A24y, d24p, preferred_element_type=jnp.float32) + cy
    pz = jnp.dot(A24z, d24p, preferred_element_type=jnp.float32) + cz
    lane = lax.broadcasted_iota(jnp.int32, (32, SPS_PAD), 1)
    pad = lane >= SPS
    px = jnp.where(pad, PAD_COORD, px)
    py = jnp.where(pad, PAD_COORD, py)
    pz = jnp.where(pad, PAD_COORD, pz)
    opx[...] = px
    opy[...] = py
    opz[...] = pz
    opn[...] = px * px + py * py + pz * pz
    cenx = jnp.dot(A24x, dbar, preferred_element_type=jnp.float32) + cx
    ceny = jnp.dot(A24y, dbar, preferred_element_type=jnp.float32) + cy
    cenz = jnp.dot(A24z, dbar, preferred_element_type=jnp.float32) + cz
    r8 = lax.broadcasted_iota(jnp.int32, (S, S), 0)
    c8 = lax.broadcasted_iota(jnp.int32, (S, S), 1)
    offdiag = (r8 != c8).astype(jnp.float32)
    ones81 = jnp.ones((S, 1), jnp.float32)
    rep_total = jnp.float32(0.0)
    dn = (((1,), (1,)), ((), ()))
    for bi in range(B):
        C = jnp.concatenate(
            [cenx[bi * S:(bi + 1) * S, :],
             ceny[bi * S:(bi + 1) * S, :],
             cenz[bi * S:(bi + 1) * S, :]], axis=1)
        G = lax.dot_general(C, C, dn, preferred_element_type=jnp.float32)
        cn = jnp.sum(C * C, axis=1, keepdims=True)
        cnT = lax.dot_general(ones81, cn, dn,
                              preferred_element_type=jnp.float32)
        d2c = jnp.maximum(cn + cnT - 2.0 * G, 0.0)
        dist = jnp.sqrt(d2c + 1e-12)
        rep = jnp.exp(5.0 * jnp.maximum(0.5 - dist, 0.0)) * offdiag
        rep_total = rep_total + jnp.sum(rep) / jnp.float32(S * (S - 1))
    rep_mean = rep_total / jnp.float32(B)
    lane128 = lax.broadcasted_iota(jnp.int32, (1, 128), 1)
    orep[...] = jnp.where(lane128 == 0, rep_mean, 0.0)


def _chamfer_kernel(tref, pxref, pyref, pzref, pnref, out,
                    colmin, acc):
    nb = pl.program_id(1)
    t = tref[0]
    tx = t[:, 0:1]
    ty = t[:, 1:2]
    tz = t[:, 2:3]
    tn = tx * tx + ty * ty + tz * tz
    px = pxref[0]
    py = pyref[0]
    pz = pzref[0]
    pn = pnref[0]
    cross = tx * px + ty * py + tz * pz
    d2 = jnp.maximum(tn + pn - 2.0 * cross, 0.0)

    bmin = jnp.min(d2, axis=0, keepdims=True)

    @pl.when(nb == 0)
    def _():
        colmin[...] = bmin
        acc[0] = 0.0

    @pl.when(nb != 0)
    def _():
        colmin[...] = jnp.minimum(colmin[...], bmin)

    m1 = jnp.min(d2, axis=1, keepdims=True)
    eq1 = d2 <= m1
    c1 = jnp.sum(eq1.astype(jnp.float32), axis=1, keepdims=True)
    d2b = jnp.where(eq1, BIGF, d2)
    m2 = jnp.min(d2b, axis=1, keepdims=True)
    eq2 = d2b <= m2
    c2 = jnp.sum(eq2.astype(jnp.float32), axis=1, keepdims=True)
    m3 = jnp.min(jnp.where(eq2, BIGF, d2b), axis=1, keepdims=True)
    k1 = jnp.minimum(c1, 3.0)
    k2 = jnp.minimum(c2, 3.0 - k1)
    k3 = 3.0 - k1 - k2
    t3 = m1 * k1 + m2 * k2 + m3 * k3
    acc[0] = acc[0] + jnp.sum(t3)

    @pl.when(nb == NB - 1)
    def _():
        lane = lax.broadcasted_iota(jnp.int32, (1, M_PAD), 1)
        valid = (lane % SPS_PAD) < SPS
        ssum = jnp.sum(jnp.where(valid, colmin[...], 0.0))
        lane128 = lax.broadcasted_iota(jnp.int32, (1, 128), 1)
        g = acc[0]
        out[...] = (jnp.where(lane128 == 0, g, 0.0)
                    + jnp.where(lane128 == 1, ssum, 0.0))


def kernel(scales, transforms, prototype_weights, prototype_offsets,
           target_pcls, sphere_verts):
    f32 = jnp.float32
    ang = transforms[..., 3:].reshape(B * S * P, 3)
    trn = transforms[..., :3].reshape(B * S * P, 3)
    a0 = ang[:, 0].reshape(32, 8)
    a1 = ang[:, 1].reshape(32, 8)
    a2 = ang[:, 2].reshape(32, 8)
    t0 = trn[:, 0].reshape(32, 8)
    t1 = trn[:, 1].reshape(32, 8)
    t2 = trn[:, 2].reshape(32, 8)
    w = prototype_weights.reshape(32, 8)
    scl = jnp.broadcast_to(scales.reshape(B, S, 1, 1), (B, S, P, 1)).reshape(32, 8)
    offs24 = prototype_offsets.transpose(2, 0, 1).reshape(24, NUM_VERTS)
    sph24 = jnp.broadcast_to(sphere_verts.T[:, None, :],
                             (3, P, NUM_VERTS)).reshape(24, NUM_VERTS)

    px, py, pz, pn, rep = pl.pallas_call(
        _prep_kernel,
        out_shape=[
            jax.ShapeDtypeStruct((32, SPS_PAD), f32),
            jax.ShapeDtypeStruct((32, SPS_PAD), f32),
            jax.ShapeDtypeStruct((32, SPS_PAD), f32),
            jax.ShapeDtypeStruct((32, SPS_PAD), f32),
            jax.ShapeDtypeStruct((1, 128), f32),
        ],
    )(a0, a1, a2, t0, t1, t2, w, scl, offs24, sph24)

    px = px.reshape(B, 1, M_PAD)
    py = py.reshape(B, 1, M_PAD)
    pz = pz.reshape(B, 1, M_PAD)
    pn = pn.reshape(B, 1, M_PAD)

    plane_spec = pl.BlockSpec((1, 1, M_PAD), lambda b, nb: (b, 0, 0))
    out2 = pl.pallas_call(
        _chamfer_kernel,
        grid=(B, NB),
        in_specs=[
            pl.BlockSpec((1, BN, 3), lambda b, nb: (b, nb, 0)),
            plane_spec, plane_spec, plane_spec, plane_spec,
        ],
        out_specs=pl.BlockSpec((1, 128), lambda b, nb: (b, 0)),
        out_shape=jax.ShapeDtypeStruct((B, 128), f32),
        scratch_shapes=[
            pltpu.VMEM((1, M_PAD), f32),
            pltpu.SMEM((1,), f32),
        ],
    )(target_pcls, px, py, pz, pn)

    g_total = jnp.sum(out2[:, 0])
    s_total = jnp.sum(out2[:, 1])
    global_loss = g_total / f32(B * N * K)
    per_slot_loss = s_total / f32(SPS) / f32(B * S)
    rep_loss = rep[0, 0]
    return 0.7 * global_loss + 0.3 * per_slot_loss + 0.2 * rep_loss


# speedup vs baseline: 49.7473x; 49.7473x over previous
import functools
import jax
import jax.numpy as jnp
from jax import lax
from jax.experimental import pallas as pl
from jax.experimental.pallas import tpu as pltpu

NUM_VERTS = 2562
SPS = 500
SPS_PAD = 512
B, S, P = 4, 8, 8
N = 4096
M_PAD = S * SPS_PAD
K = 3
BN = 256
NB = N // BN
PAD_COORD = 3.0e4
BIGF = 3.0e38


def _prep_kernel(a0, a1, a2, t0, t1, t2, w, scl, offs24, sph24,
                 opx, opy, opz, opn, orep):
    a, b, c = a0[...], a1[...], a2[...]
    ca, sa = jnp.cos(a), jnp.sin(a)
    cb, sb = jnp.cos(b), jnp.sin(b)
    cc, sc_ = jnp.cos(c), jnp.sin(c)
    R00 = cb * cc
    R01 = -cb * sc_
    R02 = sb
    R10 = sa * sb * cc + ca * sc_
    R11 = -sa * sb * sc_ + ca * cc
    R12 = -sa * cb
    R20 = -ca * sb * cc + sa * sc_
    R21 = ca * sb * sc_ + sa * cc
    R22 = ca * cb
    wv = w[...]
    ws = wv * scl[...]
    cx = jnp.sum(wv * t0[...], axis=1, keepdims=True)
    cy = jnp.sum(wv * t1[...], axis=1, keepdims=True)
    cz = jnp.sum(wv * t2[...], axis=1, keepdims=True)
    A24x = jnp.concatenate([ws * R00, ws * R01, ws * R02], axis=1)
    A24y = jnp.concatenate([ws * R10, ws * R11, ws * R12], axis=1)
    A24z = jnp.concatenate([ws * R20, ws * R21, ws * R22], axis=1)
    d24 = offs24[...] + sph24[...]
    dbar = jnp.mean(d24, axis=1, keepdims=True)
    d24p = d24[:, :SPS_PAD]
    px = jnp.dot(A24x, d24p, preferred_element_type=jnp.float32) + cx
    py = jnp.dot(A24y, d24p, preferred_element_type=jnp.float32) + cy
    pz = jnp.dot(A24z, d24p, preferred_element_type=jnp.float32) + cz
    lane = lax.broadcasted_iota(jnp.int32, (32, SPS_PAD), 1)
    pad = lane >= SPS
    px = jnp.where(pad, PAD_COORD, px)
    py = jnp.where(pad, PAD_COORD, py)
    pz = jnp.where(pad, PAD_COORD, pz)
    opx[...] = px
    opy[...] = py
    opz[...] = pz
    opn[...] = px * px + py * py + pz * pz
    cenx = jnp.dot(A24x, dbar, preferred_element_type=jnp.float32) + cx
    ceny = jnp.dot(A24y, dbar, preferred_element_type=jnp.float32) + cy
    cenz = jnp.dot(A24z, dbar, preferred_element_type=jnp.float32) + cz
    r8 = lax.broadcasted_iota(jnp.int32, (S, S), 0)
    c8 = lax.broadcasted_iota(jnp.int32, (S, S), 1)
    offdiag = (r8 != c8).astype(jnp.float32)
    ones81 = jnp.ones((S, 1), jnp.float32)
    rep_total = jnp.float32(0.0)
    dn = (((1,), (1,)), ((), ()))
    for bi in range(B):
        C = jnp.concatenate(
            [cenx[bi * S:(bi + 1) * S, :],
             ceny[bi * S:(bi + 1) * S, :],
             cenz[bi * S:(bi + 1) * S, :]], axis=1)
        G = lax.dot_general(C, C, dn, preferred_element_type=jnp.float32)
        cn = jnp.sum(C * C, axis=1, keepdims=True)
        cnT = lax.dot_general(ones81, cn, dn,
                              preferred_element_type=jnp.float32)
        d2c = jnp.maximum(cn + cnT - 2.0 * G, 0.0)
        dist = jnp.sqrt(d2c + 1e-12)
        rep = jnp.exp(5.0 * jnp.maximum(0.5 - dist, 0.0)) * offdiag
        rep_total = rep_total + jnp.sum(rep) / jnp.float32(S * (S - 1))
    rep_mean = rep_total / jnp.float32(B)
    lane128 = lax.broadcasted_iota(jnp.int32, (1, 128), 1)
    orep[...] = jnp.where(lane128 == 0, rep_mean, 0.0)


def _chamfer_kernel(tref, pxref, pyref, pzref, pnref, out,
                    colmin, acc):
    nb = pl.program_id(1)
    t = tref[0]
    tx = t[:, 0:1]
    ty = t[:, 1:2]
    tz = t[:, 2:3]
    tn = tx * tx + ty * ty + tz * tz
    px = pxref[0]
    py = pyref[0]
    pz = pzref[0]
    pn = pnref[0]
    cross = tx * px + ty * py + tz * pz
    d2 = jnp.maximum(tn + pn - 2.0 * cross, 0.0)

    bmin = jnp.min(d2, axis=0, keepdims=True)

    @pl.when(nb == 0)
    def _():
        colmin[...] = bmin
        acc[0] = 0.0

    @pl.when(nb != 0)
    def _():
        colmin[...] = jnp.minimum(colmin[...], bmin)

    m1 = jnp.min(d2, axis=1, keepdims=True)
    eq1 = d2 <= m1
    c1 = jnp.sum(eq1.astype(jnp.float32), axis=1, keepdims=True)
    d2b = jnp.where(eq1, BIGF, d2)
    m2 = jnp.min(d2b, axis=1, keepdims=True)
    eq2 = d2b <= m2
    c2 = jnp.sum(eq2.astype(jnp.float32), axis=1, keepdims=True)
    m3 = jnp.min(jnp.where(eq2, BIGF, d2b), axis=1, keepdims=True)
    k1 = jnp.minimum(c1, 3.0)
    k2 = jnp.minimum(c2, 3.0 - k1)
    k3 = 3.0 - k1 - k2
    t3 = m1 * k1 + m2 * k2 + m3 * k3
    acc[0] = acc[0] + jnp.sum(t3)

    @pl.when(nb == NB - 1)
    def _():
        lane = lax.broadcasted_iota(jnp.int32, (1, M_PAD), 1)
        valid = (lane % SPS_PAD) < SPS
        ssum = jnp.sum(jnp.where(valid, colmin[...], 0.0))
        lane128 = lax.broadcasted_iota(jnp.int32, (1, 8, 128), 2)
        g = acc[0]
        out[...] = (jnp.where(lane128 == 0, g, 0.0)
                    + jnp.where(lane128 == 1, ssum, 0.0))


def kernel(scales, transforms, prototype_weights, prototype_offsets,
           target_pcls, sphere_verts):
    f32 = jnp.float32
    ang = transforms[..., 3:].reshape(B * S * P, 3)
    trn = transforms[..., :3].reshape(B * S * P, 3)
    a0 = ang[:, 0].reshape(32, 8)
    a1 = ang[:, 1].reshape(32, 8)
    a2 = ang[:, 2].reshape(32, 8)
    t0 = trn[:, 0].reshape(32, 8)
    t1 = trn[:, 1].reshape(32, 8)
    t2 = trn[:, 2].reshape(32, 8)
    w = prototype_weights.reshape(32, 8)
    scl = jnp.broadcast_to(scales.reshape(B, S, 1, 1), (B, S, P, 1)).reshape(32, 8)
    offs24 = prototype_offsets.transpose(2, 0, 1).reshape(24, NUM_VERTS)
    sph24 = jnp.broadcast_to(sphere_verts.T[:, None, :],
                             (3, P, NUM_VERTS)).reshape(24, NUM_VERTS)

    px, py, pz, pn, rep = pl.pallas_call(
        _prep_kernel,
        out_shape=[
            jax.ShapeDtypeStruct((32, SPS_PAD), f32),
            jax.ShapeDtypeStruct((32, SPS_PAD), f32),
            jax.ShapeDtypeStruct((32, SPS_PAD), f32),
            jax.ShapeDtypeStruct((32, SPS_PAD), f32),
            jax.ShapeDtypeStruct((1, 128), f32),
        ],
    )(a0, a1, a2, t0, t1, t2, w, scl, offs24, sph24)

    px = px.reshape(B, 1, M_PAD)
    py = py.reshape(B, 1, M_PAD)
    pz = pz.reshape(B, 1, M_PAD)
    pn = pn.reshape(B, 1, M_PAD)

    plane_spec = pl.BlockSpec((1, 1, M_PAD), lambda b, nb: (b, 0, 0))
    out2 = pl.pallas_call(
        _chamfer_kernel,
        grid=(B, NB),
        in_specs=[
            pl.BlockSpec((1, BN, 3), lambda b, nb: (b, nb, 0)),
            plane_spec, plane_spec, plane_spec, plane_spec,
        ],
        out_specs=pl.BlockSpec((1, 8, 128), lambda b, nb: (b, 0, 0)),
        out_shape=jax.ShapeDtypeStruct((B, 8, 128), f32),
        scratch_shapes=[
            pltpu.VMEM((1, M_PAD), f32),
            pltpu.SMEM((1,), f32),
        ],
    )(target_pcls, px, py, pz, pn)

    g_total = jnp.sum(out2[:, 0, 0])
    s_total = jnp.sum(out2[:, 0, 1])
    global_loss = g_total / f32(B * N * K)
    per_slot_loss = s_total / f32(SPS) / f32(B * S)
    rep_loss = rep[0, 0]
    return 0.7 * global_loss + 0.3 * per_slot_loss + 0.2 * rep_loss
